# Initial kernel scaffold; baseline (speedup 1.0000x reference)
import jax
import jax.numpy as jnp
from jax.experimental import pallas as pl


def kernel(fea, pred, queues):
    raise NotImplementedError("write your pallas kernel here")


# speedup vs baseline: 1.0904x; 1.0904x over previous
import jax
import jax.numpy as jnp
from jax.experimental import pallas as pl
from jax.experimental.pallas import tpu as pltpu

_NCLS = 6
_CH = 256
_QLEN = 50000
_TEMP = 0.2
_JB = 2000
_NBLK = _QLEN // _JB


def _keys_kernel(fea_ref, pred_ref, gT_ref, cnt_ref):
    b = pl.program_id(0)
    fea = fea_ref[0]
    pred = pred_ref[0]
    best = pred[0:1, :]
    bidx = jnp.zeros_like(best, dtype=jnp.int32)
    for k in range(1, _NCLS):
        cur = pred[k:k + 1, :]
        gt = cur > best
        best = jnp.where(gt, cur, best)
        bidx = jnp.where(gt, k, bidx)
    cols = []
    cnts = []
    for c in range(_NCLS):
        m = bidx == c
        cnts.append(jnp.sum(jnp.where(m, 1.0, 0.0), axis=1, keepdims=True))
        masked = jnp.where(m, fea, 0.0)
        cols.append(jnp.sum(masked, axis=1, keepdims=True))
    keys_part = jnp.concatenate(cols, axis=1)
    cnt_part = jnp.concatenate(cnts, axis=1)

    @pl.when(b == 0)
    def _():
        gT_ref[...] = keys_part
        cnt_ref[...] = cnt_part

    @pl.when(b > 0)
    def _():
        gT_ref[...] = gT_ref[...] + keys_part
        cnt_ref[...] = cnt_ref[...] + cnt_part

    @pl.when(b == pl.num_programs(0) - 1)
    def _():
        ksum = gT_ref[...]
        cnt = cnt_ref[...]
        mean = ksum / jnp.where(cnt > 0, cnt, 1.0)
        nrm = jnp.sqrt(jnp.sum(mean * mean, axis=0, keepdims=True))
        gT_ref[...] = mean / jnp.where(nrm > 0, nrm, 1.0) / _TEMP


def _loss_kernel(q_ref, gT_ref, cnt_ref, out_ref, acc_ref, fc_ref):
    j = pl.program_id(0)
    S = q_ref[0, :, 0, 0, :]
    for c in range(1, _NCLS):
        S = S + q_ref[c, :, 0, 0, :]
    for c in range(_NCLS):
        qc = q_ref[c, :, 0, 0, :]
        g = gT_ref[:, c:c + 1]
        pos = g * qc
        neg = g * S - pos
        sh = 5.0 * jnp.abs(g)
        e = jnp.exp(pos - sh) + jnp.exp(neg - sh)

        @pl.when(j == 0)
        def _():
            acc_ref[c] = e
            fc_ref[:, c:c + 1] = pos[:, 0:1]

        @pl.when(j > 0)
        def _():
            acc_ref[c] = acc_ref[c] + e

    @pl.when(j == pl.num_programs(0) - 1)
    def _():
        cols = []
        for c in range(_NCLS):
            g = gT_ref[:, c:c + 1]
            sh = 5.0 * jnp.abs(g)
            rs = jnp.sum(acc_ref[c], axis=1, keepdims=True)
            cols.append(sh + jnp.log(rs) - fc_ref[:, c:c + 1])
        vals = jnp.concatenate(cols, axis=1)
        present = cnt_ref[...] > 0
        w = jnp.where(present, vals, 0.0)
        tot = jnp.sum(jnp.sum(w, axis=0, keepdims=True), axis=1, keepdims=True)
        out_ref[...] = tot / _CH


def kernel(fea, pred, queues):
    bs = fea.shape[0]
    hw = fea.shape[2] * fea.shape[3]
    fea_r = fea.reshape(bs, _CH, hw)
    pred_r = pred.reshape(bs, _NCLS, hw)
    gT, cnt = pl.pallas_call(
        _keys_kernel,
        grid=(bs,),
        in_specs=[
            pl.BlockSpec((1, _CH, hw), lambda b: (b, 0, 0)),
            pl.BlockSpec((1, _NCLS, hw), lambda b: (b, 0, 0)),
        ],
        out_specs=[
            pl.BlockSpec((_CH, _NCLS), lambda b: (0, 0)),
            pl.BlockSpec((1, _NCLS), lambda b: (0, 0)),
        ],
        out_shape=[
            jax.ShapeDtypeStruct((_CH, _NCLS), jnp.float32),
            jax.ShapeDtypeStruct((1, _NCLS), jnp.float32),
        ],
    )(fea_r, pred_r)
    queues_r = queues.reshape(_NCLS, _CH, _NBLK, 1, _JB)
    loss = pl.pallas_call(
        _loss_kernel,
        grid=(_NBLK,),
        in_specs=[
            pl.BlockSpec((_NCLS, _CH, 1, 1, _JB), lambda j: (0, 0, j, 0, 0)),
            pl.BlockSpec((_CH, _NCLS), lambda j: (0, 0)),
            pl.BlockSpec((1, _NCLS), lambda j: (0, 0)),
        ],
        out_specs=pl.BlockSpec((1, 1), lambda j: (0, 0)),
        out_shape=jax.ShapeDtypeStruct((1, 1), jnp.float32),
        scratch_shapes=[
            pltpu.VMEM((_NCLS, _CH, _JB), jnp.float32),
            pltpu.VMEM((_CH, _NCLS), jnp.float32),
        ],
    )(queues_r, gT, cnt)
    return loss[0, 0]


# speedup vs baseline: 2.4688x; 2.2641x over previous
import jax
import jax.numpy as jnp
from jax.experimental import pallas as pl
from jax.experimental.pallas import tpu as pltpu

_NCLS = 6
_CH = 256
_QLEN = 50000
_TEMP = 0.2
_JB = 2000
_NBLK = _QLEN // _JB


def _keys_kernel(fea_ref, pred_ref, gT_ref, cnt_ref):
    b = pl.program_id(0)
    fea = fea_ref[0]
    pred = pred_ref[0]
    best = pred[0:1, :]
    bidx = jnp.zeros_like(best, dtype=jnp.int32)
    for k in range(1, _NCLS):
        cur = pred[k:k + 1, :]
        gt = cur > best
        best = jnp.where(gt, cur, best)
        bidx = jnp.where(gt, k, bidx)
    cols = []
    cnts = []
    for c in range(_NCLS):
        m = bidx == c
        cnts.append(jnp.sum(jnp.where(m, 1.0, 0.0), axis=1, keepdims=True))
        masked = jnp.where(m, fea, 0.0)
        cols.append(jnp.sum(masked, axis=1, keepdims=True))
    keys_part = jnp.concatenate(cols, axis=1)
    cnt_part = jnp.concatenate(cnts, axis=1)

    @pl.when(b == 0)
    def _():
        gT_ref[...] = keys_part
        cnt_ref[...] = cnt_part

    @pl.when(b > 0)
    def _():
        gT_ref[...] = gT_ref[...] + keys_part
        cnt_ref[...] = cnt_ref[...] + cnt_part

    @pl.when(b == pl.num_programs(0) - 1)
    def _():
        ksum = gT_ref[...]
        cnt = cnt_ref[...]
        mean = ksum / jnp.where(cnt > 0, cnt, 1.0)
        nrm = jnp.sqrt(jnp.sum(mean * mean, axis=0, keepdims=True))
        gT_ref[...] = mean / jnp.where(nrm > 0, nrm, 1.0) / _TEMP


def _loss_kernel(q_ref, gT_ref, cnt_ref, out_ref):
    k = pl.program_id(0)
    S = q_ref[0]
    for c in range(1, _NCLS):
        S = S + q_ref[c]
    cols = []
    for c in range(_NCLS):
        qc = q_ref[c]
        g = gT_ref[:, c:c + 1]
        a1 = g * qc
        a2 = g * (S - qc)
        es = jnp.exp(a1) + jnp.exp(a2)
        rs = jnp.sum(es, axis=1, keepdims=True)
        cols.append(jnp.log(rs) - a1[:, 0:1])
    vals = jnp.concatenate(cols, axis=1)
    present = cnt_ref[...] > 0
    w = jnp.where(present, vals, 0.0)
    part = jnp.sum(jnp.sum(w, axis=0, keepdims=True), axis=1, keepdims=True)

    @pl.when(k == 0)
    def _():
        out_ref[...] = part / _CH

    @pl.when(k > 0)
    def _():
        out_ref[...] = out_ref[...] + part / _CH


def kernel(fea, pred, queues):
    bs = fea.shape[0]
    hw = fea.shape[2] * fea.shape[3]
    fea_r = fea.reshape(bs, _CH, hw)
    pred_r = pred.reshape(bs, _NCLS, hw)
    gT, cnt = pl.pallas_call(
        _keys_kernel,
        grid=(bs,),
        in_specs=[
            pl.BlockSpec((1, _CH, hw), lambda b: (b, 0, 0)),
            pl.BlockSpec((1, _NCLS, hw), lambda b: (b, 0, 0)),
        ],
        out_specs=[
            pl.BlockSpec((_CH, _NCLS), lambda b: (0, 0)),
            pl.BlockSpec((1, _NCLS), lambda b: (0, 0)),
        ],
        out_shape=[
            jax.ShapeDtypeStruct((_CH, _NCLS), jnp.float32),
            jax.ShapeDtypeStruct((1, _NCLS), jnp.float32),
        ],
    )(fea_r, pred_r)
    chunk = 8
    loss = pl.pallas_call(
        _loss_kernel,
        grid=(_CH // chunk,),
        in_specs=[
            pl.BlockSpec((_NCLS, chunk, _QLEN), lambda k: (0, k, 0)),
            pl.BlockSpec((chunk, _NCLS), lambda k: (k, 0)),
            pl.BlockSpec((1, _NCLS), lambda k: (0, 0)),
        ],
        out_specs=pl.BlockSpec((1, 1), lambda k: (0, 0)),
        out_shape=jax.ShapeDtypeStruct((1, 1), jnp.float32),
    )(queues, gT, cnt)
    return loss[0, 0]


# speedup vs baseline: 6.1229x; 2.4801x over previous
import jax
import jax.numpy as jnp
from jax.experimental import pallas as pl
from jax.experimental.pallas import tpu as pltpu

_NCLS = 6
_CH = 256
_QLEN = 50000
_TEMP = 0.2
_JQ = 2000
_NBLK = _QLEN // _JQ


def _keys_kernel(fea_ref, pred_ref, gM_ref, cnt_ref):
    b = pl.program_id(0)
    fea = fea_ref[0]
    pred = pred_ref[0]
    best = pred[0:1, :]
    bidx = jnp.zeros_like(best, dtype=jnp.int32)
    for k in range(1, _NCLS):
        cur = pred[k:k + 1, :]
        gt = cur > best
        best = jnp.where(gt, cur, best)
        bidx = jnp.where(gt, k, bidx)
    cls = jax.lax.broadcasted_iota(jnp.int32, (_NCLS, pred.shape[1]), 0)
    masks = (cls == bidx).astype(jnp.float32)
    cnt_part = jnp.sum(masks, axis=1, keepdims=True)
    keys_part = jax.lax.dot_general(
        masks, fea, (((1,), (0,)), ((), ())),
        preferred_element_type=jnp.float32,
        precision=jax.lax.Precision.HIGHEST)

    @pl.when(b == 0)
    def _():
        gM_ref[...] = keys_part
        cnt_ref[...] = cnt_part

    @pl.when(b > 0)
    def _():
        gM_ref[...] = gM_ref[...] + keys_part
        cnt_ref[...] = cnt_ref[...] + cnt_part

    @pl.when(b == pl.num_programs(0) - 1)
    def _():
        ksum = gM_ref[...]
        cnt = cnt_ref[...]
        mean = ksum / jnp.where(cnt > 0, cnt, 1.0)
        nrm = jnp.sqrt(jnp.sum(mean * mean, axis=1, keepdims=True))
        gM_ref[...] = mean / jnp.where(nrm > 0, nrm, 1.0) / _TEMP


def _loss_kernel(q_ref, gM_ref, cnt_ref, out_ref, acc_ref, fc_ref):
    k = pl.program_id(0)
    S = q_ref[0]
    for c in range(1, _NCLS):
        S = S + q_ref[c]
    for c in range(_NCLS):
        qc = q_ref[c]
        g = gM_ref[c:c + 1, :]
        a1 = g * qc
        a2 = g * (S - qc)
        es = jnp.exp(a1) + jnp.exp(a2)
        red = jnp.sum(es, axis=0, keepdims=True)

        @pl.when(k == 0)
        def _():
            acc_ref[c:c + 1, :] = red
            fc_ref[c:c + 1, :] = a1[0:1, :]

        @pl.when(k > 0)
        def _():
            acc_ref[c:c + 1, :] = acc_ref[c:c + 1, :] + red

    @pl.when(k == pl.num_programs(0) - 1)
    def _():
        vals = jnp.log(acc_ref[...]) - fc_ref[...]
        ce = jnp.sum(vals, axis=1, keepdims=True) / _CH
        w = jnp.where(cnt_ref[...] > 0, ce, 0.0)
        out_ref[...] = jnp.sum(w, axis=0, keepdims=True)


def kernel(fea, pred, queues):
    bs = fea.shape[0]
    hw = fea.shape[2] * fea.shape[3]
    fea_t = fea.transpose(0, 2, 3, 1).reshape(bs, hw, _CH)
    pred_r = pred.reshape(bs, _NCLS, hw)
    q_t = queues.transpose(0, 2, 1)
    gM, cnt = pl.pallas_call(
        _keys_kernel,
        grid=(bs,),
        in_specs=[
            pl.BlockSpec((1, hw, _CH), lambda b: (b, 0, 0)),
            pl.BlockSpec((1, _NCLS, hw), lambda b: (b, 0, 0)),
        ],
        out_specs=[
            pl.BlockSpec((_NCLS, _CH), lambda b: (0, 0)),
            pl.BlockSpec((_NCLS, 1), lambda b: (0, 0)),
        ],
        out_shape=[
            jax.ShapeDtypeStruct((_NCLS, _CH), jnp.float32),
            jax.ShapeDtypeStruct((_NCLS, 1), jnp.float32),
        ],
    )(fea_t, pred_r)
    loss = pl.pallas_call(
        _loss_kernel,
        grid=(_NBLK,),
        in_specs=[
            pl.BlockSpec((_NCLS, _JQ, _CH), lambda k: (0, k, 0)),
            pl.BlockSpec((_NCLS, _CH), lambda k: (0, 0)),
            pl.BlockSpec((_NCLS, 1), lambda k: (0, 0)),
        ],
        out_specs=pl.BlockSpec((1, 1), lambda k: (0, 0)),
        out_shape=jax.ShapeDtypeStruct((1, 1), jnp.float32),
        scratch_shapes=[
            pltpu.VMEM((_NCLS, _CH), jnp.float32),
            pltpu.VMEM((_NCLS, _CH), jnp.float32),
        ],
    )(q_t, gM, cnt)
    return loss[0, 0]


# speedup vs baseline: 6.7950x; 1.1098x over previous
import jax
import jax.numpy as jnp
from jax.experimental import pallas as pl
from jax.experimental.pallas import tpu as pltpu

_NCLS = 6
_CH = 256
_QLEN = 50000
_TEMP = 0.2
_JQ = 2000
_NBLK = _QLEN // _JQ


def _keys_kernel(fea_ref, pred_ref, gM_ref, cnt_ref):
    b = pl.program_id(0)
    fea = fea_ref[0]
    pred = pred_ref[0]
    best = pred[0:1, :]
    bidx = jnp.zeros_like(best, dtype=jnp.int32)
    for k in range(1, _NCLS):
        cur = pred[k:k + 1, :]
        gt = cur > best
        best = jnp.where(gt, cur, best)
        bidx = jnp.where(gt, k, bidx)
    cls = jax.lax.broadcasted_iota(jnp.int32, (_NCLS, pred.shape[1]), 0)
    masks = (cls == bidx).astype(jnp.float32)
    cnt_part = jnp.sum(masks, axis=1, keepdims=True)
    keys_part = jax.lax.dot_general(
        masks, fea, (((1,), (0,)), ((), ())),
        preferred_element_type=jnp.float32,
        precision=jax.lax.Precision.HIGHEST)

    @pl.when(b == 0)
    def _():
        gM_ref[...] = keys_part
        cnt_ref[...] = cnt_part

    @pl.when(b > 0)
    def _():
        gM_ref[...] = gM_ref[...] + keys_part
        cnt_ref[...] = cnt_ref[...] + cnt_part

    @pl.when(b == pl.num_programs(0) - 1)
    def _():
        ksum = gM_ref[...]
        cnt = cnt_ref[...]
        mean = ksum / jnp.where(cnt > 0, cnt, 1.0)
        nrm = jnp.sqrt(jnp.sum(mean * mean, axis=1, keepdims=True))
        gM_ref[...] = mean / jnp.where(nrm > 0, nrm, 1.0) / _TEMP


def _loss_kernel(q_ref, gM_ref, cnt_ref, out_ref, acc_ref, fc_ref):
    k = pl.program_id(0)
    g = [gM_ref[c:c + 1, :] for c in range(_NCLS)]
    rows = 8

    def body(i, carry):
        base = i * rows
        q = [q_ref[c, pl.ds(base, rows), :] for c in range(_NCLS)]
        s = q[0]
        for c in range(1, _NCLS):
            s = s + q[c]
        out = []
        for c in range(_NCLS):
            a1 = g[c] * q[c]
            a2 = g[c] * (s - q[c])
            out.append(carry[c] + jnp.exp(a1) + jnp.exp(a2))
        return tuple(out)

    init = tuple(jnp.zeros((rows, _CH), jnp.float32) for _ in range(_NCLS))
    accs = jax.lax.fori_loop(0, _JQ // rows, body, init, unroll=2)
    for c in range(_NCLS):
        red = jnp.sum(accs[c], axis=0, keepdims=True)

        @pl.when(k == 0)
        def _():
            acc_ref[c:c + 1, :] = red
            fc_ref[c:c + 1, :] = g[c] * q_ref[c, 0:1, :]

        @pl.when(k > 0)
        def _():
            acc_ref[c:c + 1, :] = acc_ref[c:c + 1, :] + red

    @pl.when(k == pl.num_programs(0) - 1)
    def _():
        vals = jnp.log(acc_ref[...]) - fc_ref[...]
        ce = jnp.sum(vals, axis=1, keepdims=True) / _CH
        w = jnp.where(cnt_ref[...] > 0, ce, 0.0)
        out_ref[...] = jnp.sum(w, axis=0, keepdims=True)


def kernel(fea, pred, queues):
    bs = fea.shape[0]
    hw = fea.shape[2] * fea.shape[3]
    fea_t = fea.transpose(0, 2, 3, 1).reshape(bs, hw, _CH)
    pred_r = pred.reshape(bs, _NCLS, hw)
    q_t = queues.transpose(0, 2, 1)
    gM, cnt = pl.pallas_call(
        _keys_kernel,
        grid=(bs,),
        in_specs=[
            pl.BlockSpec((1, hw, _CH), lambda b: (b, 0, 0)),
            pl.BlockSpec((1, _NCLS, hw), lambda b: (b, 0, 0)),
        ],
        out_specs=[
            pl.BlockSpec((_NCLS, _CH), lambda b: (0, 0)),
            pl.BlockSpec((_NCLS, 1), lambda b: (0, 0)),
        ],
        out_shape=[
            jax.ShapeDtypeStruct((_NCLS, _CH), jnp.float32),
            jax.ShapeDtypeStruct((_NCLS, 1), jnp.float32),
        ],
    )(fea_t, pred_r)
    loss = pl.pallas_call(
        _loss_kernel,
        grid=(_NBLK,),
        in_specs=[
            pl.BlockSpec((_NCLS, _JQ, _CH), lambda k: (0, k, 0)),
            pl.BlockSpec((_NCLS, _CH), lambda k: (0, 0)),
            pl.BlockSpec((_NCLS, 1), lambda k: (0, 0)),
        ],
        out_specs=pl.BlockSpec((1, 1), lambda k: (0, 0)),
        out_shape=jax.ShapeDtypeStruct((1, 1), jnp.float32),
        scratch_shapes=[
            pltpu.VMEM((_NCLS, _CH), jnp.float32),
            pltpu.VMEM((_NCLS, _CH), jnp.float32),
        ],
    )(q_t, gM, cnt)
    return loss[0, 0]


# speedup vs baseline: 7.6570x; 1.1268x over previous
import jax
import jax.numpy as jnp
from jax.experimental import pallas as pl
from jax.experimental.pallas import tpu as pltpu

_NCLS = 6
_CH = 256
_QLEN = 50000
_TEMP = 0.2
_JQ = 2000
_NBLK = _QLEN // _JQ


def _keys_kernel(fea_ref, pred_ref, gM_ref, cnt_ref):
    b = pl.program_id(0)
    fea = fea_ref[0]
    pred = pred_ref[0]
    best = pred[0:1, :]
    bidx = jnp.zeros_like(best, dtype=jnp.int32)
    for k in range(1, _NCLS):
        cur = pred[k:k + 1, :]
        gt = cur > best
        best = jnp.where(gt, cur, best)
        bidx = jnp.where(gt, k, bidx)
    cls = jax.lax.broadcasted_iota(jnp.int32, (_NCLS, pred.shape[1]), 0)
    masks = (cls == bidx).astype(jnp.float32)
    cnt_part = jnp.sum(masks, axis=1, keepdims=True)
    keys_part = jax.lax.dot_general(
        masks, fea, (((1,), (0,)), ((), ())),
        preferred_element_type=jnp.float32,
        precision=jax.lax.Precision.HIGHEST)

    @pl.when(b == 0)
    def _():
        gM_ref[...] = keys_part
        cnt_ref[...] = cnt_part

    @pl.when(b > 0)
    def _():
        gM_ref[...] = gM_ref[...] + keys_part
        cnt_ref[...] = cnt_ref[...] + cnt_part

    @pl.when(b == pl.num_programs(0) - 1)
    def _():
        ksum = gM_ref[...]
        cnt = cnt_ref[...]
        mean = ksum / jnp.where(cnt > 0, cnt, 1.0)
        nrm = jnp.sqrt(jnp.sum(mean * mean, axis=1, keepdims=True))
        gM_ref[...] = mean / jnp.where(nrm > 0, nrm, 1.0) / _TEMP


def _loss_kernel(q_ref, gM_ref, cnt_ref, out_ref, acc_ref, fc_ref):
    k = pl.program_id(0)
    g = [gM_ref[c:c + 1, :] for c in range(_NCLS)]
    rows = 8

    def body(i, carry):
        base = i * rows
        q = [q_ref[c, pl.ds(base, rows), :] for c in range(_NCLS)]
        s = q[0]
        for c in range(1, _NCLS):
            s = s + q[c]
        out = []
        for c in range(_NCLS):
            a1 = g[c] * q[c]
            a2 = g[c] * (s - q[c])
            out.append(carry[c] + jnp.exp(a1) + jnp.exp(a2))
        return tuple(out)

    init = tuple(jnp.zeros((rows, _CH), jnp.float32) for _ in range(_NCLS))
    accs = jax.lax.fori_loop(0, _JQ // rows, body, init, unroll=4)
    for c in range(_NCLS):
        red = jnp.sum(accs[c], axis=0, keepdims=True)

        @pl.when(k == 0)
        def _():
            acc_ref[c:c + 1, :] = red
            fc_ref[c:c + 1, :] = g[c] * q_ref[c, 0:1, :]

        @pl.when(k > 0)
        def _():
            acc_ref[c:c + 1, :] = acc_ref[c:c + 1, :] + red

    @pl.when(k == pl.num_programs(0) - 1)
    def _():
        vals = jnp.log(acc_ref[...]) - fc_ref[...]
        ce = jnp.sum(vals, axis=1, keepdims=True) / _CH
        w = jnp.where(cnt_ref[...] > 0, ce, 0.0)
        out_ref[...] = jnp.sum(w, axis=0, keepdims=True)


def kernel(fea, pred, queues):
    bs = fea.shape[0]
    hw = fea.shape[2] * fea.shape[3]
    fea_t = fea.transpose(0, 2, 3, 1).reshape(bs, hw, _CH)
    pred_r = pred.reshape(bs, _NCLS, hw)
    q_t = queues.transpose(0, 2, 1)
    gM, cnt = pl.pallas_call(
        _keys_kernel,
        grid=(bs,),
        in_specs=[
            pl.BlockSpec((1, hw, _CH), lambda b: (b, 0, 0)),
            pl.BlockSpec((1, _NCLS, hw), lambda b: (b, 0, 0)),
        ],
        out_specs=[
            pl.BlockSpec((_NCLS, _CH), lambda b: (0, 0)),
            pl.BlockSpec((_NCLS, 1), lambda b: (0, 0)),
        ],
        out_shape=[
            jax.ShapeDtypeStruct((_NCLS, _CH), jnp.float32),
            jax.ShapeDtypeStruct((_NCLS, 1), jnp.float32),
        ],
    )(fea_t, pred_r)
    loss = pl.pallas_call(
        _loss_kernel,
        grid=(_NBLK,),
        in_specs=[
            pl.BlockSpec((_NCLS, _JQ, _CH), lambda k: (0, k, 0)),
            pl.BlockSpec((_NCLS, _CH), lambda k: (0, 0)),
            pl.BlockSpec((_NCLS, 1), lambda k: (0, 0)),
        ],
        out_specs=pl.BlockSpec((1, 1), lambda k: (0, 0)),
        out_shape=jax.ShapeDtypeStruct((1, 1), jnp.float32),
        scratch_shapes=[
            pltpu.VMEM((_NCLS, _CH), jnp.float32),
            pltpu.VMEM((_NCLS, _CH), jnp.float32),
        ],
    )(q_t, gM, cnt)
    return loss[0, 0]


# speedup vs baseline: 7.8667x; 1.0274x over previous
import jax
import jax.numpy as jnp
from jax.experimental import pallas as pl
from jax.experimental.pallas import tpu as pltpu

_NCLS = 6
_CH = 256
_QLEN = 50000
_TEMP = 0.2
_JQ = 2000
_NBLK = _QLEN // _JQ


def _keys_kernel(fea_ref, pred_ref, gM_ref, cnt_ref):
    b = pl.program_id(0)
    fea = fea_ref[0]
    pred = pred_ref[0]
    best = pred[0:1, :]
    bidx = jnp.zeros_like(best, dtype=jnp.int32)
    for k in range(1, _NCLS):
        cur = pred[k:k + 1, :]
        gt = cur > best
        best = jnp.where(gt, cur, best)
        bidx = jnp.where(gt, k, bidx)
    cls = jax.lax.broadcasted_iota(jnp.int32, (_NCLS, pred.shape[1]), 0)
    masks = (cls == bidx).astype(jnp.float32)
    cnt_part = jnp.sum(masks, axis=1, keepdims=True)
    keys_part = jax.lax.dot_general(
        masks, fea, (((1,), (0,)), ((), ())),
        preferred_element_type=jnp.float32,
        precision=jax.lax.Precision.HIGHEST)

    @pl.when(b == 0)
    def _():
        gM_ref[...] = keys_part
        cnt_ref[...] = cnt_part

    @pl.when(b > 0)
    def _():
        gM_ref[...] = gM_ref[...] + keys_part
        cnt_ref[...] = cnt_ref[...] + cnt_part

    @pl.when(b == pl.num_programs(0) - 1)
    def _():
        ksum = gM_ref[...]
        cnt = cnt_ref[...]
        mean = ksum / jnp.where(cnt > 0, cnt, 1.0)
        nrm = jnp.sqrt(jnp.sum(mean * mean, axis=1, keepdims=True))
        gM_ref[...] = mean / jnp.where(nrm > 0, nrm, 1.0) / _TEMP


def _loss_kernel(q_ref, gM_ref, cnt_ref, out_ref, acc_ref, fc_ref):
    k = pl.program_id(0)
    g = [gM_ref[c:c + 1, :] for c in range(_NCLS)]
    rows = 8
    gb = [jnp.tile(gc, (rows, 1)) for gc in g]

    def body(i, carry):
        base = i * rows
        q = [q_ref[c, pl.ds(base, rows), :] for c in range(_NCLS)]
        s = (q[0] + q[1]) + (q[2] + q[3]) + (q[4] + q[5])
        out = []
        for c in range(_NCLS):
            a1 = gb[c] * q[c]
            a2 = gb[c] * (s - q[c])
            out.append(carry[c] + jnp.exp(a1) + jnp.exp(a2))
        return tuple(out)

    init = tuple(jnp.zeros((rows, _CH), jnp.float32) for _ in range(_NCLS))
    accs = jax.lax.fori_loop(0, _JQ // rows, body, init, unroll=4)
    for c in range(_NCLS):
        red = jnp.sum(accs[c], axis=0, keepdims=True)

        @pl.when(k == 0)
        def _():
            acc_ref[c:c + 1, :] = red
            fc_ref[c:c + 1, :] = g[c] * q_ref[c, 0:1, :]

        @pl.when(k > 0)
        def _():
            acc_ref[c:c + 1, :] = acc_ref[c:c + 1, :] + red

    @pl.when(k == pl.num_programs(0) - 1)
    def _():
        vals = jnp.log(acc_ref[...]) - fc_ref[...]
        ce = jnp.sum(vals, axis=1, keepdims=True) / _CH
        w = jnp.where(cnt_ref[...] > 0, ce, 0.0)
        out_ref[...] = jnp.sum(w, axis=0, keepdims=True)


def kernel(fea, pred, queues):
    bs = fea.shape[0]
    hw = fea.shape[2] * fea.shape[3]
    fea_t = fea.transpose(0, 2, 3, 1).reshape(bs, hw, _CH)
    pred_r = pred.reshape(bs, _NCLS, hw)
    q_t = queues.transpose(0, 2, 1)
    gM, cnt = pl.pallas_call(
        _keys_kernel,
        grid=(bs,),
        in_specs=[
            pl.BlockSpec((1, hw, _CH), lambda b: (b, 0, 0)),
            pl.BlockSpec((1, _NCLS, hw), lambda b: (b, 0, 0)),
        ],
        out_specs=[
            pl.BlockSpec((_NCLS, _CH), lambda b: (0, 0)),
            pl.BlockSpec((_NCLS, 1), lambda b: (0, 0)),
        ],
        out_shape=[
            jax.ShapeDtypeStruct((_NCLS, _CH), jnp.float32),
            jax.ShapeDtypeStruct((_NCLS, 1), jnp.float32),
        ],
    )(fea_t, pred_r)
    loss = pl.pallas_call(
        _loss_kernel,
        grid=(_NBLK,),
        in_specs=[
            pl.BlockSpec((_NCLS, _JQ, _CH), lambda k: (0, k, 0)),
            pl.BlockSpec((_NCLS, _CH), lambda k: (0, 0)),
            pl.BlockSpec((_NCLS, 1), lambda k: (0, 0)),
        ],
        out_specs=pl.BlockSpec((1, 1), lambda k: (0, 0)),
        out_shape=jax.ShapeDtypeStruct((1, 1), jnp.float32),
        scratch_shapes=[
            pltpu.VMEM((_NCLS, _CH), jnp.float32),
            pltpu.VMEM((_NCLS, _CH), jnp.float32),
        ],
    )(q_t, gM, cnt)
    return loss[0, 0]
